# baseline (speedup 1.0000x reference)
import functools

import jax
import jax.numpy as jnp
from jax import lax
from jax.experimental import pallas as pl
from jax.experimental.pallas import tpu as pltpu
from jax.experimental.pallas import tpu_sc as plsc

_ROWS = 16384
_COLS = 200
_D = 16
_NW = 32
_IW = _ROWS // _NW
_ISEG = 256
_JT = 8
_NUNIT = (_COLS // _JT) * (_IW // _ISEG)

_GDN = lax.GatherDimensionNumbers(
    offset_dims=(), collapsed_slice_dims=(0,), start_index_map=(0,))


def _vgather(src, idx):
    return lax.gather(src, idx[:, None], _GDN, slice_sizes=(1,),
                      mode=lax.GatherScatterMode.PROMISE_IN_BOUNDS)


@functools.partial(
    pl.kernel,
    mesh=plsc.VectorSubcoreMesh(core_axis_name="c", subcore_axis_name="s"),
    compiler_params=pltpu.CompilerParams(needs_layout_passes=False),
    out_type=jax.ShapeDtypeStruct((_COLS, _D, _ROWS), jnp.float32),
    scratch_types=[
        pltpu.VMEM((256,), jnp.float32),
        pltpu.VMEM((_JT, _ISEG), jnp.int32),
        pltpu.VMEM((_JT, _ISEG), jnp.int32),
        pltpu.VMEM((_JT, _D, _ISEG), jnp.float32),
        pltpu.VMEM((_JT, _D, _ISEG), jnp.float32),
        pltpu.SemaphoreType.DMA,
        pltpu.SemaphoreType.DMA,
        pltpu.SemaphoreType.DMA,
        pltpu.SemaphoreType.DMA,
    ],
)
def _emb_lookup(tab_hbm, idxt_hbm, out_hbm,
                tab_v, idx0, idx1, blk0, blk1, si0, si1, so0, so1):
    wid = lax.axis_index("s") * 2 + lax.axis_index("c")
    i_lo = wid * _IW
    idxs = (idx0, idx1)
    blks = (blk0, blk1)
    sis = (si0, si1)
    sos = (so0, so1)

    pltpu.sync_copy(tab_hbm, tab_v)
    tcol = [tab_v[pl.ds(k * 16, 16)] for k in range(_D)]

    def unit_slices(u):
        uc = lax.min(u, _NUNIT - 1)
        jt = uc // 2
        i0 = i_lo + (uc % 2) * _ISEG
        return pl.ds(jt * _JT, _JT), pl.ds(i0, _ISEG)

    def idx_fetch(u, b):
        js, is_ = unit_slices(u)
        pltpu.async_copy(idxt_hbm.at[js, is_], idxs[b], sis[b])

    idx_fetch(0, 0)
    idx_fetch(1, 1)

    def unit_body(u, carry):
        for b in range(2):
            uu = u * 2 + b
            js, is_ = unit_slices(uu)
            pltpu.make_async_copy(
                idxt_hbm.at[js, is_], idxs[b], sis[b]).wait()

            @pl.when(u > 0)
            def _():
                pltpu.make_async_copy(
                    blks[b], out_hbm.at[js, :, is_], sos[b]).wait()

            idx_v, blk = idxs[b], blks[b]

            def j_body(j, c1):
                def iv_body(iv, c2):
                    idxv = idx_v[j, pl.ds(iv * 16, 16)]
                    for k in range(_D):
                        blk[j, k, pl.ds(iv * 16, 16)] = _vgather(
                            tcol[k], idxv)
                    return c2
                lax.fori_loop(0, _ISEG // 16, iv_body, 0, unroll=4)
                return c1
            lax.fori_loop(0, _JT, j_body, 0, unroll=2)

            pltpu.async_copy(blk, out_hbm.at[js, :, is_], sos[b])
            idx_fetch(uu + 2, b)
        return carry

    lax.fori_loop(0, _NUNIT // 2, unit_body, 0)

    for b in range(2):
        js, is_ = unit_slices(_NUNIT - 2 + b)
        pltpu.make_async_copy(
            idxt_hbm.at[js, is_], idxs[b], sis[b]).wait()
        pltpu.make_async_copy(
            blks[b], out_hbm.at[js, :, is_], sos[b]).wait()


def kernel(regime_id, table):
    idx_t = jnp.transpose(regime_id).astype(jnp.int32)
    tab_t = jnp.pad(jnp.transpose(table), ((0, 0), (0, 12)))
    out_t = _emb_lookup(tab_t.reshape(256), idx_t)
    return jnp.transpose(out_t, (2, 0, 1))


# speedup vs baseline: 1.0155x; 1.0155x over previous
import functools

import jax
import jax.numpy as jnp
from jax import lax
from jax.experimental import pallas as pl
from jax.experimental.pallas import tpu as pltpu
from jax.experimental.pallas import tpu_sc as plsc

_ROWS = 16384
_COLS = 200
_D = 16
_NW = 32
_IW = _ROWS // _NW
_ISEG = 256
_JT = 8
_NUNIT = (_COLS // _JT) * (_IW // _ISEG)
_NB = 3
_NTRIP = (_NUNIT + _NB - 1) // _NB

_GDN = lax.GatherDimensionNumbers(
    offset_dims=(), collapsed_slice_dims=(0,), start_index_map=(0,))


def _vgather(src, idx):
    return lax.gather(src, idx[:, None], _GDN, slice_sizes=(1,),
                      mode=lax.GatherScatterMode.PROMISE_IN_BOUNDS)


@functools.partial(
    pl.kernel,
    mesh=plsc.VectorSubcoreMesh(core_axis_name="c", subcore_axis_name="s"),
    compiler_params=pltpu.CompilerParams(needs_layout_passes=False),
    out_type=jax.ShapeDtypeStruct((_COLS, _D, _ROWS), jnp.float32),
    scratch_types=(
        [pltpu.VMEM((256,), jnp.float32)]
        + [pltpu.VMEM((_JT, _ISEG), jnp.int32) for _ in range(_NB)]
        + [pltpu.VMEM((_JT, _D, _ISEG), jnp.float32) for _ in range(_NB)]
        + [pltpu.SemaphoreType.DMA for _ in range(2 * _NB)]
    ),
)
def _emb_lookup(tab_hbm, idxt_hbm, out_hbm, tab_v, *bufs):
    idxs = bufs[0:_NB]
    blks = bufs[_NB:2 * _NB]
    sis = bufs[2 * _NB:3 * _NB]
    sos = bufs[3 * _NB:4 * _NB]
    wid = lax.axis_index("s") * 2 + lax.axis_index("c")
    i_lo = wid * _IW

    pltpu.sync_copy(tab_hbm, tab_v)
    tcol = [tab_v[pl.ds(k * 16, 16)] for k in range(_D)]

    def unit_slices(u):
        uc = lax.min(u, _NUNIT - 1)
        jt = uc // 2
        i0 = i_lo + (uc % 2) * _ISEG
        return pl.ds(jt * _JT, _JT), pl.ds(i0, _ISEG)

    def idx_fetch(u, b):
        js, is_ = unit_slices(u)
        pltpu.async_copy(idxt_hbm.at[js, is_], idxs[b], sis[b])

    for b in range(_NB):
        idx_fetch(b, b)

    def trip_body(t, carry):
        for b in range(_NB):
            uu = t * _NB + b
            js, is_ = unit_slices(uu)
            pltpu.make_async_copy(
                idxt_hbm.at[js, is_], idxs[b], sis[b]).wait()

            @pl.when(t > 0)
            def _():
                pltpu.make_async_copy(
                    blks[b], out_hbm.at[js, :, is_], sos[b]).wait()

            idx_v, blk = idxs[b], blks[b]

            def j_body(j, c1):
                def iv_body(iv, c2):
                    idxv = idx_v[j, pl.ds(iv * 16, 16)]
                    for k in range(_D):
                        blk[j, k, pl.ds(iv * 16, 16)] = _vgather(
                            tcol[k], idxv)
                    return c2
                lax.fori_loop(0, _ISEG // 16, iv_body, 0, unroll=2)
                return c1
            lax.fori_loop(0, _JT, j_body, 0)

            pltpu.async_copy(blk, out_hbm.at[js, :, is_], sos[b])
            idx_fetch(uu + _NB, b)
        return carry

    lax.fori_loop(0, _NTRIP, trip_body, 0)

    for b in range(_NB):
        js, is_ = unit_slices((_NTRIP - 1) * _NB + b)
        pltpu.make_async_copy(
            idxt_hbm.at[js, is_], idxs[b], sis[b]).wait()
        pltpu.make_async_copy(
            blks[b], out_hbm.at[js, :, is_], sos[b]).wait()


def kernel(regime_id, table):
    idx_t = jnp.transpose(regime_id).astype(jnp.int32)
    tab_t = jnp.pad(jnp.transpose(table), ((0, 0), (0, 12)))
    out_t = _emb_lookup(tab_t.reshape(256), idx_t)
    return jnp.transpose(out_t, (2, 0, 1))


# speedup vs baseline: 1.0309x; 1.0151x over previous
import functools

import jax
import jax.numpy as jnp
from jax import lax
from jax.experimental import pallas as pl
from jax.experimental.pallas import tpu as pltpu
from jax.experimental.pallas import tpu_sc as plsc

_ROWS = 16384
_COLS = 200
_D = 16
_NW = 32
_IW = _ROWS // _NW
_ISEG = 256
_JT = 8
_NUNIT = (_COLS // _JT) * (_IW // _ISEG)

_GDN = lax.GatherDimensionNumbers(
    offset_dims=(), collapsed_slice_dims=(0,), start_index_map=(0,))


def _vgather(src, idx):
    return lax.gather(src, idx[:, None], _GDN, slice_sizes=(1,),
                      mode=lax.GatherScatterMode.PROMISE_IN_BOUNDS)


@functools.partial(
    pl.kernel,
    mesh=plsc.VectorSubcoreMesh(core_axis_name="c", subcore_axis_name="s"),
    compiler_params=pltpu.CompilerParams(needs_layout_passes=False),
    out_type=jax.ShapeDtypeStruct((_COLS, _D, _ROWS), jnp.float32),
    scratch_types=[
        pltpu.VMEM((256,), jnp.float32),
        pltpu.VMEM((_JT, _ISEG), jnp.int32),
        pltpu.VMEM((_JT, _ISEG), jnp.int32),
        pltpu.VMEM((_JT, _D, _ISEG), jnp.float32),
        pltpu.VMEM((_JT, _D, _ISEG), jnp.float32),
        pltpu.SemaphoreType.DMA,
        pltpu.SemaphoreType.DMA,
        pltpu.SemaphoreType.DMA,
        pltpu.SemaphoreType.DMA,
    ],
)
def _emb_lookup(tab_hbm, idxt_hbm, out_hbm,
                tab_v, idx0, idx1, blk0, blk1, si0, si1, so0, so1):
    wid = lax.axis_index("s") * 2 + lax.axis_index("c")
    i_lo = wid * _IW
    idxs = (idx0, idx1)
    blks = (blk0, blk1)
    sis = (si0, si1)
    sos = (so0, so1)

    pltpu.sync_copy(tab_hbm, tab_v)
    tcol = [tab_v[pl.ds(k * 16, 16)] for k in range(_D)]

    def unit_slices(u):
        uc = lax.min(u, _NUNIT - 1)
        jt = uc // 2
        i0 = i_lo + (uc % 2) * _ISEG
        return pl.ds(jt * _JT, _JT), pl.ds(i0, _ISEG)

    def idx_fetch(u, b):
        js, is_ = unit_slices(u)
        pltpu.async_copy(idxt_hbm.at[js, is_], idxs[b], sis[b])

    idx_fetch(0, 0)
    idx_fetch(1, 1)

    def unit_body(u, carry):
        for b in range(2):
            uu = u * 2 + b
            js, is_ = unit_slices(uu)
            pltpu.make_async_copy(
                idxt_hbm.at[js, is_], idxs[b], sis[b]).wait()

            @pl.when(u > 0)
            def _():
                pltpu.make_async_copy(
                    blks[b], out_hbm.at[js, :, is_], sos[b]).wait()

            idx_v, blk = idxs[b], blks[b]

            def j_body(j, c1):
                def iv_body(iv, c2):
                    idxv = idx_v[j, pl.ds(iv * 16, 16)]
                    for k in range(_D):
                        blk[j, k, pl.ds(iv * 16, 16)] = _vgather(
                            tcol[k], idxv)
                    return c2
                lax.fori_loop(0, _ISEG // 16, iv_body, 0, unroll=2)
                return c1
            lax.fori_loop(0, _JT, j_body, 0)

            pltpu.async_copy(blk, out_hbm.at[js, :, is_], sos[b])
            idx_fetch(uu + 2, b)
        return carry

    lax.fori_loop(0, _NUNIT // 2, unit_body, 0)

    for b in range(2):
        js, is_ = unit_slices(_NUNIT - 2 + b)
        pltpu.make_async_copy(
            idxt_hbm.at[js, is_], idxs[b], sis[b]).wait()
        pltpu.make_async_copy(
            blks[b], out_hbm.at[js, :, is_], sos[b]).wait()


def kernel(regime_id, table):
    idx_t = jnp.transpose(regime_id).astype(jnp.int32)
    tab_t = jnp.pad(jnp.transpose(table), ((0, 0), (0, 12)))
    out_t = _emb_lookup(tab_t.reshape(256), idx_t)
    return jnp.transpose(out_t, (2, 0, 1))


# speedup vs baseline: 1.0344x; 1.0034x over previous
import functools

import jax
import jax.numpy as jnp
from jax import lax
from jax.experimental import pallas as pl
from jax.experimental.pallas import tpu as pltpu
from jax.experimental.pallas import tpu_sc as plsc

_ROWS = 16384
_COLS = 200
_D = 16
_NW = 32
_IW = _ROWS // _NW
_ISEG = 512
_JT = 4
_NSEG = _IW // _ISEG
_NUNIT = (_COLS // _JT) * _NSEG

_GDN = lax.GatherDimensionNumbers(
    offset_dims=(), collapsed_slice_dims=(0,), start_index_map=(0,))


def _vgather(src, idx):
    return lax.gather(src, idx[:, None], _GDN, slice_sizes=(1,),
                      mode=lax.GatherScatterMode.PROMISE_IN_BOUNDS)


@functools.partial(
    pl.kernel,
    mesh=plsc.VectorSubcoreMesh(core_axis_name="c", subcore_axis_name="s"),
    compiler_params=pltpu.CompilerParams(needs_layout_passes=False),
    out_type=jax.ShapeDtypeStruct((_COLS, _D, _ROWS), jnp.float32),
    scratch_types=[
        pltpu.VMEM((256,), jnp.float32),
        pltpu.VMEM((_JT, _ISEG), jnp.int32),
        pltpu.VMEM((_JT, _ISEG), jnp.int32),
        pltpu.VMEM((_JT, _D, _ISEG), jnp.float32),
        pltpu.VMEM((_JT, _D, _ISEG), jnp.float32),
        pltpu.SemaphoreType.DMA,
        pltpu.SemaphoreType.DMA,
        pltpu.SemaphoreType.DMA,
        pltpu.SemaphoreType.DMA,
    ],
)
def _emb_lookup(tab_hbm, idxt_hbm, out_hbm,
                tab_v, idx0, idx1, blk0, blk1, si0, si1, so0, so1):
    wid = lax.axis_index("s") * 2 + lax.axis_index("c")
    i_lo = wid * _IW
    idxs = (idx0, idx1)
    blks = (blk0, blk1)
    sis = (si0, si1)
    sos = (so0, so1)

    pltpu.sync_copy(tab_hbm, tab_v)
    tcol = [tab_v[pl.ds(k * 16, 16)] for k in range(_D)]

    def unit_slices(u):
        uc = lax.min(u, _NUNIT - 1)
        jt = uc // _NSEG
        i0 = i_lo + (uc % _NSEG) * _ISEG
        return pl.ds(jt * _JT, _JT), pl.ds(i0, _ISEG)

    def idx_fetch(u, b):
        js, is_ = unit_slices(u)
        pltpu.async_copy(idxt_hbm.at[js, is_], idxs[b], sis[b])

    idx_fetch(0, 0)
    idx_fetch(1, 1)

    def unit_body(u, carry):
        for b in range(2):
            uu = u * 2 + b
            js, is_ = unit_slices(uu)
            pltpu.make_async_copy(
                idxt_hbm.at[js, is_], idxs[b], sis[b]).wait()

            @pl.when(u > 0)
            def _():
                pltpu.make_async_copy(
                    blks[b], out_hbm.at[js, :, is_], sos[b]).wait()

            idx_v, blk = idxs[b], blks[b]

            def j_body(j, c1):
                def iv_body(iv, c2):
                    idxv = idx_v[j, pl.ds(iv * 16, 16)]
                    for k in range(_D):
                        blk[j, k, pl.ds(iv * 16, 16)] = _vgather(
                            tcol[k], idxv)
                    return c2
                lax.fori_loop(0, _ISEG // 16, iv_body, 0, unroll=2)
                return c1
            lax.fori_loop(0, _JT, j_body, 0)

            pltpu.async_copy(blk, out_hbm.at[js, :, is_], sos[b])
            idx_fetch(uu + 2, b)
        return carry

    lax.fori_loop(0, _NUNIT // 2, unit_body, 0)

    for b in range(2):
        js, is_ = unit_slices(_NUNIT - 2 + b)
        pltpu.make_async_copy(
            idxt_hbm.at[js, is_], idxs[b], sis[b]).wait()
        pltpu.make_async_copy(
            blks[b], out_hbm.at[js, :, is_], sos[b]).wait()


def kernel(regime_id, table):
    idx_t = jnp.transpose(regime_id).astype(jnp.int32)
    tab_t = jnp.pad(jnp.transpose(table), ((0, 0), (0, 12)))
    out_t = _emb_lookup(tab_t.reshape(256), idx_t)
    return jnp.transpose(out_t, (2, 0, 1))
